# baseline (speedup 1.0000x reference)
import functools

import jax
import jax.numpy as jnp
from jax import lax
from jax.experimental import pallas as pl
from jax.experimental.pallas import tpu as pltpu
from jax.experimental.pallas import tpu_sc as plsc


def kernel(labels, train, table):
    del train
    B = labels.shape[0]
    V, D = table.shape
    H = V // 2
    info = plsc.get_sparse_core_info()
    NC, NS = info.num_cores, info.num_subcores
    NW = NC * NS
    b_per_w = B // NW

    C = 128
    NCH = b_per_w // C
    RG = 16

    mesh = plsc.VectorSubcoreMesh(core_axis_name="c", subcore_axis_name="s")

    @functools.partial(
        pl.kernel,
        mesh=mesh,
        compiler_params=pltpu.CompilerParams(use_tc_tiling_on_sc=False),
        out_type=jax.ShapeDtypeStruct((B, D), jnp.float32),
        scratch_types=[
            pltpu.VMEM((b_per_w,), jnp.int32),
            pltpu.VMEM((NCH, C), jnp.int32),
            pltpu.VMEM((NCH, C), jnp.int32),
            pltpu.VMEM((NCH, C, D), jnp.float32),
            pltpu.VMEM((NCH, C, D), jnp.float32),
            pltpu.VMEM((2, C, D), jnp.float32),
            pltpu.SemaphoreType.DMA((NCH,)),
            pltpu.SemaphoreType.DMA((NCH,)),
            pltpu.SemaphoreType.DMA((2,)),
        ],
    )
    def emb(t0_hbm, t1_hbm, idx_hbm, out_hbm, idx_s, idx0_v, idx1_v,
            buf0_v, buf1_v, comp_v, g0sems, g1sems, psems):
        wid = lax.axis_index("s") * NC + lax.axis_index("c")
        base = wid * b_per_w
        pltpu.sync_copy(idx_hbm.at[wid], idx_s)

        for jc in range(NCH):
            for v in range(C // 16):
                l = idx_s[pl.ds(jc * C + v * 16, 16)]
                idx0_v[jc, pl.ds(v * 16, 16)] = jnp.minimum(l, H - 1)
                idx1_v[jc, pl.ds(v * 16, 16)] = jnp.minimum(
                    jnp.maximum(l - H, 0), (V - H) - 1
                )

        gets = []
        for jc in range(NCH):
            c0 = pltpu.async_copy(
                t0_hbm.at[idx0_v.at[jc]], buf0_v.at[jc], g0sems.at[jc]
            )
            c1 = pltpu.async_copy(
                t1_hbm.at[idx1_v.at[jc]], buf1_v.at[jc], g1sems.at[jc]
            )
            gets.append((c0, c1))

        def wait_writeback(cb):
            pltpu.make_async_copy(
                comp_v.at[cb], out_hbm.at[pl.ds(base, C)], psems.at[cb]
            ).wait()

        for jc in range(NCH):
            cb = jc % 2
            gets[jc][0].wait()
            gets[jc][1].wait()
            if jc >= 2:
                wait_writeback(cb)

            @pl.loop(0, C // RG)
            def _(rg):
                lvec = idx_s[pl.ds(jc * C + rg * RG, RG)]
                for r in range(RG):
                    i = rg * RG + r
                    sel = lvec[r] >= H
                    for k in range(D // 16):
                        a = buf0_v[jc, i, pl.ds(k * 16, 16)]
                        b = buf1_v[jc, i, pl.ds(k * 16, 16)]
                        comp_v[cb, i, pl.ds(k * 16, 16)] = jnp.where(
                            sel, b, a
                        )

            pltpu.async_copy(
                comp_v.at[cb], out_hbm.at[pl.ds(base + jc * C, C)],
                psems.at[cb],
            )

        wait_writeback(0)
        wait_writeback(1)

    return emb(table[:H], table[H:], labels.reshape(NW, b_per_w))


# speedup vs baseline: 2.3745x; 2.3745x over previous
import functools

import jax
import jax.numpy as jnp
from jax import lax
from jax.experimental import pallas as pl
from jax.experimental.pallas import tpu as pltpu
from jax.experimental.pallas import tpu_sc as plsc


def kernel(labels, train, table):
    del train
    B = labels.shape[0]
    V, D = table.shape
    info = plsc.get_sparse_core_info()
    NC, NS = info.num_cores, info.num_subcores
    NW = NC * NS
    b_per_w = B // NW

    C = 32
    NG = b_per_w // C

    mesh = plsc.VectorSubcoreMesh(core_axis_name="c", subcore_axis_name="s")

    @functools.partial(
        pl.kernel,
        mesh=mesh,
        out_type=jax.ShapeDtypeStruct((B, D), jnp.float32),
        scratch_types=[
            pltpu.VMEM((b_per_w,), jnp.int32),
            pltpu.VMEM((2, C, D), jnp.float32),
            pltpu.SemaphoreType.DMA((2,)),
            pltpu.SemaphoreType.DMA((2,)),
        ],
    )
    def emb(table_hbm, idx_hbm, out_hbm, idx_s, rows_v, gsems, psems):
        wid = lax.axis_index("s") * NC + lax.axis_index("c")
        base = wid * b_per_w
        pltpu.sync_copy(idx_hbm.at[wid], idx_s)

        def issue_group(g, buf):
            cps = []
            for v in range(C // 16):
                vec = idx_s[pl.ds(g * C + v * 16, 16)]
                for s in range(16):
                    i = vec[s]
                    cps.append(
                        pltpu.async_copy(
                            table_hbm.at[pl.ds(i, 1)],
                            rows_v.at[buf].at[pl.ds(v * 16 + s, 1)],
                            gsems.at[buf],
                        )
                    )
            return cps

        def drain_group(cps):
            for cp in cps:
                cp.wait()

        def writeback(g, buf):
            return pltpu.async_copy(
                rows_v.at[buf], out_hbm.at[pl.ds(base + g * C, C)], psems.at[buf]
            )

        def wait_writeback(buf):
            pltpu.make_async_copy(
                rows_v.at[buf], out_hbm.at[pl.ds(base, C)], psems.at[buf]
            ).wait()

        @pl.loop(0, NG // 2)
        def _(gg):
            g0 = gg * 2
            g1 = g0 + 1

            @pl.when(gg > 0)
            def _():
                wait_writeback(0)
                wait_writeback(1)

            cps0 = issue_group(g0, 0)
            cps1 = issue_group(g1, 1)
            drain_group(cps0)
            writeback(g0, 0)
            drain_group(cps1)
            writeback(g1, 1)

        wait_writeback(0)
        wait_writeback(1)

    return emb(table, labels.reshape(NW, b_per_w))
